# baseline (speedup 1.0000x reference)
import jax
import jax.numpy as jnp
from jax import lax
from jax.experimental import pallas as pl
from jax.experimental.pallas import tpu as pltpu
from jax.experimental.pallas import tpu_sc as plsc

N = 320000
D = 128
S = 10000

CHUNK = 64
NUM_CHUNKS = N // CHUNK
EXT = 144
NC = 2
NS = 16
S_PAD = 10240
SEG_PER_SUB = S_PAD // NS

SBLK = 2000

N_TC = 192000
TBLK = 512
W = 64
SC_OFF_CHUNKS = N_TC // CHUNK
NUM_SC_CHUNKS = (N - N_TC) // CHUNK


def _pool_body(v_hbm, i_hbm, gw_hbm, p_hbm,
               wbuf, dbuf, gwbuf, acc, cnt, sem0, sem1):
    c = lax.axis_index("c")
    s = lax.axis_index("s")
    pltpu.sync_copy(gw_hbm, gwbuf)

    lane = lax.iota(jnp.int32, 16)
    first = (lane == 0).astype(jnp.float32)
    gw = [gwbuf[pl.ds(k * 16, 16)] for k in range(8)]
    cnt[0] = 0

    zero16 = jnp.zeros((16,), jnp.float32)

    @pl.loop(0, CHUNK)
    def _(r):
        for k in range(9):
            wbuf[0, r, pl.ds(k * 16, 16)] = zero16

    @pl.loop(0, SEG_PER_SUB // CHUNK)
    def _(j):
        pltpu.sync_copy(
            wbuf.at[0],
            acc.at[pl.ds(s * SEG_PER_SUB + j * CHUNK, CHUNK)],
        )

    plsc.subcore_barrier()

    def chunk_body(v_vmem, i_vmem):
        n = cnt[0]

        def do(p, sem):
            wb = wbuf.at[p]

            @pl.when(n >= 2)
            def _():
                pltpu.make_async_copy(wb, acc.at[dbuf.at[0]], sem).wait()

            @plsc.parallel_loop(0, CHUNK, unroll=8)
            def _(r):
                v = [v_vmem[r, pl.ds(k * 16, 16)] for k in range(8)]
                p0 = v[0] * gw[0] + v[1] * gw[1]
                p1 = v[2] * gw[2] + v[3] * gw[3]
                p2 = v[4] * gw[4] + v[5] * gw[5]
                p3 = v[6] * gw[6] + v[7] * gw[7]
                tot = jnp.sum((p0 + p1) + (p2 + p3))
                ev = jnp.exp(lax.broadcast_in_dim(tot, (16,), ()))
                for k in range(8):
                    wb[r, pl.ds(k * 16, 16)] = v[k] * ev
                wb[r, pl.ds(D, 16)] = ev * first

            pltpu.async_copy(wb, acc.at[i_vmem.at[0]], sem, add=True)

        @pl.when(n % 2 == 0)
        def _():
            do(0, sem0)

        @pl.when(n % 2 == 1)
        def _():
            do(1, sem1)

        cnt[0] = n + 1

    pltpu.emit_pipeline(
        chunk_body,
        grid=(NUM_SC_CHUNKS,),
        in_specs=[
            pl.BlockSpec((CHUNK, D), lambda i: (i + SC_OFF_CHUNKS, 0)),
            pl.BlockSpec((1, CHUNK), lambda i: (i + SC_OFF_CHUNKS, 0)),
        ],
        core_axis_name=("c", "s"),
        dimension_semantics=(pltpu.PARALLEL,),
    )(v_hbm, i_hbm)

    pltpu.make_async_copy(wbuf.at[0], acc.at[dbuf.at[0]], sem0).wait()
    pltpu.make_async_copy(wbuf.at[1], acc.at[dbuf.at[0]], sem1).wait()
    plsc.subcore_barrier()

    @pl.loop(0, SEG_PER_SUB // CHUNK)
    def _(j):
        base = s * SEG_PER_SUB + j * CHUNK
        pltpu.sync_copy(acc.at[pl.ds(base, CHUNK)], wbuf.at[0])
        pltpu.sync_copy(wbuf.at[0], p_hbm.at[c, pl.ds(base, CHUNK)])


def _pool(values, idx2, gw):
    mesh = plsc.VectorSubcoreMesh(core_axis_name="c", subcore_axis_name="s")
    f = pl.kernel(
        _pool_body,
        out_type=jax.ShapeDtypeStruct((NC, S_PAD, EXT), jnp.float32),
        mesh=mesh,
        compiler_params=pltpu.CompilerParams(
            use_tc_tiling_on_sc=False, needs_layout_passes=False
        ),
        scratch_types=[
            pltpu.VMEM((2, CHUNK, EXT), jnp.float32),
            pltpu.VMEM((1, CHUNK), jnp.int32),
            pltpu.VMEM((D,), jnp.float32),
            pltpu.VMEM_SHARED((S_PAD, EXT), jnp.float32),
            pltpu.SMEM((1,), jnp.int32),
            pltpu.SemaphoreType.DMA,
            pltpu.SemaphoreType.DMA,
        ],
    )
    return f(values, idx2, gw)


def _tc_pool_body(v_ref, i_ref, gw_ref, o_ref):
    @pl.when(pl.program_id(0) == 0)
    def _():
        o_ref[...] = jnp.zeros((S_PAD, EXT), jnp.float32)

    vb = v_ref[...]
    idx_row = i_ref[0]
    e = jnp.exp(jnp.sum(vb * gw_ref[...], axis=1, keepdims=True))
    weighted = jnp.concatenate(
        [vb * e, e, jnp.zeros((TBLK, EXT - D - 1), jnp.float32)], axis=1
    )
    big = jnp.int32(1 << 30)

    def cond(s):
        return s < big

    def body(s):
        s0a = pl.multiple_of(s - lax.rem(s, 8), 8)
        cols = s0a + lax.broadcasted_iota(jnp.int32, (W, 1), 0)
        oht = (cols == idx_row).astype(jnp.float32)
        contrib = lax.dot_general(
            oht, weighted, (((1,), (0,)), ((), ())),
            preferred_element_type=jnp.float32,
        )
        o_ref[pl.ds(s0a, W), :] += contrib
        return jnp.min(jnp.where(idx_row >= s0a + W, idx_row, big))

    lax.while_loop(cond, body, i_ref[0, 0, 0])


_tc_pool = pl.pallas_call(
    _tc_pool_body,
    grid=(N_TC // TBLK,),
    in_specs=[
        pl.BlockSpec((TBLK, D), lambda i: (i, 0)),
        pl.BlockSpec((1, 1, TBLK), lambda i: (i, 0, 0)),
        pl.BlockSpec((1, D), lambda i: (0, 0)),
    ],
    out_specs=pl.BlockSpec((S_PAD, EXT), lambda i: (0, 0)),
    out_shape=jax.ShapeDtypeStruct((S_PAD, EXT), jnp.float32),
)


def _finish_body(p_ref, t_ref, aw_ref, ab_ref, o_ref):
    acc = p_ref[0, :, :D] + p_ref[1, :, :D] + t_ref[:, :D]
    z = (
        p_ref[0, :, D : D + 1]
        + p_ref[1, :, D : D + 1]
        + t_ref[:, D : D + 1]
    )
    pooled = acc * jnp.where(z > 0, 1.0 / z, 0.0)
    bias = jnp.where(z > 0, 1.0, 0.0) * ab_ref[...].reshape(1, D)
    o_ref[...] = (
        jnp.dot(pooled, aw_ref[...], preferred_element_type=jnp.float32) + bias
    )


_finish = pl.pallas_call(
    _finish_body,
    grid=(S // SBLK,),
    in_specs=[
        pl.BlockSpec((NC, SBLK, EXT), lambda i: (0, i, 0)),
        pl.BlockSpec((SBLK, EXT), lambda i: (i, 0)),
        pl.BlockSpec((D, D), lambda i: (0, 0)),
        pl.BlockSpec((D,), lambda i: (0,)),
    ],
    out_specs=pl.BlockSpec((SBLK, D), lambda i: (i, 0)),
    out_shape=jax.ShapeDtypeStruct((S, D), jnp.float32),
)


def kernel(values, indices, gate_w, gate_b, attn_w, attn_b):
    del gate_b
    idx32 = indices.astype(jnp.int32)
    idx2 = idx32.reshape(NUM_CHUNKS, CHUNK)
    idx3 = idx32[:N_TC].reshape(N_TC // TBLK, 1, TBLK)
    gw = gate_w.reshape(D)
    gw2 = gate_w.reshape(1, D)
    tcp = _tc_pool(values, idx3, gw2)
    partial = _pool(values, idx2, gw)
    return _finish(partial, tcp, attn_w, attn_b)


# speedup vs baseline: 1.6244x; 1.6244x over previous
import jax
import jax.numpy as jnp
from jax import lax
from jax.experimental import pallas as pl
from jax.experimental.pallas import tpu as pltpu
from jax.experimental.pallas import tpu_sc as plsc

N = 320000
D = 128
S = 10000

CHUNK = 64
NUM_CHUNKS = N // CHUNK
EXT = 144
NC = 2
NS = 16
S_PAD = 10240
SEG_PER_SUB = S_PAD // NS

SBLK = 2000


def _pool_body(v_hbm, i_hbm, gw_hbm, p_hbm,
               wbuf, dbuf, gwbuf, acc, cnt, sem0, sem1):
    c = lax.axis_index("c")
    s = lax.axis_index("s")
    pltpu.sync_copy(gw_hbm, gwbuf)

    lane = lax.iota(jnp.int32, 16)
    first = (lane == 0).astype(jnp.float32)
    gw = [gwbuf[pl.ds(k * 16, 16)] for k in range(8)]
    cnt[0] = 0

    zero16 = jnp.zeros((16,), jnp.float32)

    @pl.loop(0, CHUNK)
    def _(r):
        for k in range(9):
            wbuf[0, r, pl.ds(k * 16, 16)] = zero16

    @pl.loop(0, SEG_PER_SUB // CHUNK)
    def _(j):
        pltpu.sync_copy(
            wbuf.at[0],
            acc.at[pl.ds(s * SEG_PER_SUB + j * CHUNK, CHUNK)],
        )

    plsc.subcore_barrier()

    def chunk_body(v_vmem, i_vmem):
        n = cnt[0]

        def do(p, sem):
            wb = wbuf.at[p]

            @pl.when(n >= 2)
            def _():
                pltpu.make_async_copy(wb, acc.at[dbuf.at[0]], sem).wait()

            @plsc.parallel_loop(0, CHUNK, unroll=16)
            def _(r):
                v = [v_vmem[r, pl.ds(k * 16, 16)] for k in range(8)]
                p0 = v[0] * gw[0] + v[1] * gw[1]
                p1 = v[2] * gw[2] + v[3] * gw[3]
                p2 = v[4] * gw[4] + v[5] * gw[5]
                p3 = v[6] * gw[6] + v[7] * gw[7]
                tot = jnp.sum((p0 + p1) + (p2 + p3))
                ev = jnp.exp(lax.broadcast_in_dim(tot, (16,), ()))
                for k in range(8):
                    wb[r, pl.ds(k * 16, 16)] = v[k] * ev
                wb[r, pl.ds(D, 16)] = ev * first

            pltpu.async_copy(wb, acc.at[i_vmem.at[0]], sem, add=True)

        @pl.when(n % 2 == 0)
        def _():
            do(0, sem0)

        @pl.when(n % 2 == 1)
        def _():
            do(1, sem1)

        cnt[0] = n + 1

    pltpu.emit_pipeline(
        chunk_body,
        grid=(NUM_CHUNKS,),
        in_specs=[
            pl.BlockSpec((CHUNK, D), lambda i: (i, 0)),
            pl.BlockSpec((1, CHUNK), lambda i: (i, 0)),
        ],
        core_axis_name=("c", "s"),
        dimension_semantics=(pltpu.PARALLEL,),
    )(v_hbm, i_hbm)

    pltpu.make_async_copy(wbuf.at[0], acc.at[dbuf.at[0]], sem0).wait()
    pltpu.make_async_copy(wbuf.at[1], acc.at[dbuf.at[0]], sem1).wait()
    plsc.subcore_barrier()

    @pl.loop(0, SEG_PER_SUB // CHUNK)
    def _(j):
        base = s * SEG_PER_SUB + j * CHUNK
        pltpu.sync_copy(acc.at[pl.ds(base, CHUNK)], wbuf.at[0])
        pltpu.sync_copy(wbuf.at[0], p_hbm.at[c, pl.ds(base, CHUNK)])


def _pool(values, idx2, gw):
    mesh = plsc.VectorSubcoreMesh(core_axis_name="c", subcore_axis_name="s")
    f = pl.kernel(
        _pool_body,
        out_type=jax.ShapeDtypeStruct((NC, S_PAD, EXT), jnp.float32),
        mesh=mesh,
        compiler_params=pltpu.CompilerParams(
            use_tc_tiling_on_sc=False, needs_layout_passes=False
        ),
        scratch_types=[
            pltpu.VMEM((2, CHUNK, EXT), jnp.float32),
            pltpu.VMEM((1, CHUNK), jnp.int32),
            pltpu.VMEM((D,), jnp.float32),
            pltpu.VMEM_SHARED((S_PAD, EXT), jnp.float32),
            pltpu.SMEM((1,), jnp.int32),
            pltpu.SemaphoreType.DMA,
            pltpu.SemaphoreType.DMA,
        ],
    )
    return f(values, idx2, gw)


def _finish_body(p_ref, aw_ref, ab_ref, o_ref):
    acc = p_ref[0, :, :D] + p_ref[1, :, :D]
    z = p_ref[0, :, D : D + 1] + p_ref[1, :, D : D + 1]
    pooled = acc * jnp.where(z > 0, 1.0 / z, 0.0)
    bias = jnp.where(z > 0, 1.0, 0.0) * ab_ref[...].reshape(1, D)
    o_ref[...] = (
        jnp.dot(pooled, aw_ref[...], preferred_element_type=jnp.float32) + bias
    )


_finish = pl.pallas_call(
    _finish_body,
    grid=(S // SBLK,),
    in_specs=[
        pl.BlockSpec((NC, SBLK, EXT), lambda i: (0, i, 0)),
        pl.BlockSpec((D, D), lambda i: (0, 0)),
        pl.BlockSpec((D,), lambda i: (0,)),
    ],
    out_specs=pl.BlockSpec((SBLK, D), lambda i: (i, 0)),
    out_shape=jax.ShapeDtypeStruct((S, D), jnp.float32),
)


def kernel(values, indices, gate_w, gate_b, attn_w, attn_b):
    del gate_b
    idx2 = indices.astype(jnp.int32).reshape(NUM_CHUNKS, CHUNK)
    gw = gate_w.reshape(D)
    partial = _pool(values, idx2, gw)
    return _finish(partial, attn_w, attn_b)
